# Initial kernel scaffold; baseline (speedup 1.0000x reference)
import jax
import jax.numpy as jnp
from jax.experimental import pallas as pl


def kernel(x, token_table, pos_table):
    raise NotImplementedError("write your pallas kernel here")


# speedup vs baseline: 1.2412x; 1.2412x over previous
import jax
import jax.numpy as jnp
from jax import lax
from jax.experimental import pallas as pl
from jax.experimental.pallas import tpu as pltpu
from jax.experimental.pallas import tpu_sc as plsc

_VOCAB = 1000000
_MAXLEN = 200
_EMBED = 32
_BATCH = 4096

_NW = 32
_SEQ_PER_W = _BATCH // _NW
_ROWS_PER_W = _SEQ_PER_W * _MAXLEN
_G1 = 128
_G2 = _MAXLEN - _G1


def _sc_body(x_hbm, tok_hbm, pos_hbm, out_hbm,
             idx_all, rows0, rows1, pos_v, g0, g1, o0, o1):
    cid = lax.axis_index("c")
    sid = lax.axis_index("s")
    wid = sid * 2 + cid
    base_seq = wid * _SEQ_PER_W

    pltpu.sync_copy(x_hbm.at[pl.ds(wid * _ROWS_PER_W, _ROWS_PER_W)], idx_all)
    pltpu.sync_copy(pos_hbm, pos_v)

    rows = (rows0, rows1)
    gsem = (g0, g1)
    osem = (o0, o1)

    def fire_gather(sl, b):
        off = pl.multiple_of(sl * _MAXLEN, 8)
        pltpu.async_copy(tok_hbm.at[idx_all.at[pl.ds(off, _G1)]],
                         rows[b].at[pl.ds(0, _G1)], gsem[b])
        pltpu.async_copy(tok_hbm.at[idx_all.at[pl.ds(off + _G1, _G2)]],
                         rows[b].at[pl.ds(_G1, _G2)], gsem[b])

    def wait_gather(b):
        pltpu.make_async_copy(tok_hbm.at[idx_all.at[pl.ds(0, _G1)]],
                              rows[b].at[pl.ds(0, _G1)], gsem[b]).wait()
        pltpu.make_async_copy(tok_hbm.at[idx_all.at[pl.ds(_G1, _G2)]],
                              rows[b].at[pl.ds(_G1, _G2)], gsem[b]).wait()

    def fire_out(sl, b):
        pltpu.async_copy(rows[b], out_hbm.at[base_seq + sl], osem[b])

    def wait_out(b):
        pltpu.make_async_copy(rows[b], out_hbm.at[base_seq], osem[b]).wait()

    def add_pos(b):
        def body(i, carry):
            for k in range(_EMBED // 16):
                sl = pl.ds(k * 16, 16)
                rows[b][i, sl] = rows[b][i, sl] + pos_v[i, sl]
            return carry
        lax.fori_loop(0, _MAXLEN, body, 0, unroll=4)

    fire_gather(0, 0)

    def outer(g2, carry):
        g = g2 * 2
        for b in range(2):
            sl = g + b
            nxt = sl + 1
            nb = 1 - b

            @pl.when(nxt < _SEQ_PER_W)
            def _prefetch():
                @pl.when(nxt >= 2)
                def _drain():
                    wait_out(nb)
                fire_gather(nxt, nb)

            wait_gather(b)
            add_pos(b)
            fire_out(sl, b)
        return carry

    lax.fori_loop(0, _SEQ_PER_W // 2, outer, 0)
    wait_out(0)
    wait_out(1)


def kernel(x, token_table, pos_table):
    x_flat = x.reshape(-1).astype(jnp.int32)
    mesh = plsc.VectorSubcoreMesh(core_axis_name="c", subcore_axis_name="s")
    f = pl.kernel(
        _sc_body,
        out_type=jax.ShapeDtypeStruct((_BATCH, _MAXLEN, _EMBED), jnp.float32),
        mesh=mesh,
        compiler_params=pltpu.CompilerParams(use_tc_tiling_on_sc=False),
        scratch_types=[
            pltpu.VMEM((_ROWS_PER_W,), jnp.int32),
            pltpu.VMEM((_MAXLEN, _EMBED), jnp.float32),
            pltpu.VMEM((_MAXLEN, _EMBED), jnp.float32),
            pltpu.VMEM((_MAXLEN, _EMBED), jnp.float32),
            pltpu.SemaphoreType.DMA,
            pltpu.SemaphoreType.DMA,
            pltpu.SemaphoreType.DMA,
            pltpu.SemaphoreType.DMA,
        ],
    )
    return f(x_flat, token_table, pos_table)
